# baseline (speedup 1.0000x reference)
import functools
import math

import jax
import jax.numpy as jnp
from jax import lax
from jax.experimental import pallas as pl
from jax.experimental.pallas import tpu as pltpu
from jax.experimental.pallas import tpu_sc as plsc

HID = 1024
HEAD_DIM = 64
BASE = 10000.0
EPS = 1e-12

NC = 2
NS = 16
NW = NC * NS

TOK = 8192
TPW = TOK // NW
CH = 32
NCH = TPW // CH

_sc_mesh = plsc.VectorSubcoreMesh(
    core_axis_name="c", subcore_axis_name="s", num_cores=NC, num_subcores=NS
)


def _make_gather_sc(n_tok):
    tpw = n_tok // NW
    nch = tpw // CH

    @functools.partial(
        pl.kernel,
        out_type=jax.ShapeDtypeStruct((n_tok, HID), jnp.float32),
        mesh=_sc_mesh,
        scratch_types=[
            pltpu.VMEM((nch, CH), jnp.int32),
            pltpu.VMEM((CH, HID), jnp.float32),
            pltpu.VMEM((CH, HID), jnp.float32),
            pltpu.SemaphoreType.DMA,
            pltpu.SemaphoreType.DMA,
        ],
    )
    def _gather_sc(ids_hbm, table_hbm, out_hbm, idx_v, buf_a, buf_b, sem_a, sem_b):
        wid = lax.axis_index("s") * NC + lax.axis_index("c")
        pltpu.sync_copy(ids_hbm.at[wid], idx_v)
        bufs = (buf_a, buf_b)
        sems = (sem_a, sem_b)

        def start(c):
            pltpu.make_async_copy(
                table_hbm.at[idx_v.at[c]], bufs[c % 2], sems[c % 2]
            ).start()

        def wait(c):
            pltpu.make_async_copy(
                table_hbm.at[idx_v.at[c]], bufs[c % 2], sems[c % 2]
            ).wait()

        start(0)
        for c in range(nch):
            if c + 1 < nch:
                start(c + 1)
            wait(c)
            pltpu.sync_copy(bufs[c % 2], out_hbm.at[pl.ds(wid * tpw + c * CH, CH)])

    return _gather_sc


def _ln_math(x, g, b):
    mu = jnp.mean(x, axis=1, keepdims=True)
    xc = x - mu
    var = jnp.mean(xc * xc, axis=1, keepdims=True)
    return xc * lax.rsqrt(var + EPS) * g + b


def _ln_body(rows_ref, tt_ref, g_ref, b_ref, out_ref):
    out_ref[...] = _ln_math(rows_ref[...] + tt_ref[...], g_ref[...], b_ref[...])


def _ln_body_acc(rows_ref, tt_ref, g_ref, b_ref, prev_ref, out_ref):
    del prev_ref
    out_ref[...] = _ln_math(rows_ref[...] + tt_ref[...], g_ref[...], b_ref[...])


TB = 1024


def _ln_tc_seg(rows, tt0, gamma, beta, seg, prev):
    n_tok = rows.shape[0]
    nblk = n_tok // TB
    off = seg * nblk
    row_spec = pl.BlockSpec((TB, HID), lambda i: (i, 0))
    chan_spec = pl.BlockSpec((1, HID), lambda i: (0, 0))
    out_spec = pl.BlockSpec((TB, HID), lambda i, o=off: (o + i, 0))
    args = [rows, tt0.reshape(1, HID), gamma.reshape(1, HID), beta.reshape(1, HID)]
    in_specs = [row_spec, chan_spec, chan_spec, chan_spec]
    body = _ln_body
    kwargs = {}
    if seg > 0:
        args.append(prev)
        in_specs.append(pl.BlockSpec(memory_space=pl.ANY))
        body = _ln_body_acc
        kwargs["input_output_aliases"] = {4: 0}
    return pl.pallas_call(
        body,
        grid=(nblk,),
        in_specs=in_specs,
        out_specs=out_spec,
        out_shape=jax.ShapeDtypeStruct((TOK, HID), jnp.float32),
        **kwargs,
    )(*args)


def _rope_body(cos_ref, sin_ref):
    s_len, d = cos_ref.shape
    half = d // 2
    pos = lax.broadcasted_iota(jnp.int32, (s_len, half), 0).astype(jnp.float32)
    i = lax.broadcasted_iota(jnp.int32, (s_len, half), 1).astype(jnp.float32)
    inv_freq = jnp.exp(i * (-2.0 * math.log(BASE) / d))
    ang = pos * inv_freq
    c = jnp.cos(ang)
    s = jnp.sin(ang)
    cos_ref[:, :half] = c
    cos_ref[:, half:] = c
    sin_ref[:, :half] = s
    sin_ref[:, half:] = s


NSEG = 2
SEG = TOK // NSEG
_gather_seg = _make_gather_sc(SEG)


def kernel(input_ids, word_emb, token_type_emb, ln_gamma, ln_beta):
    b, s = input_ids.shape
    ids = input_ids.reshape(NSEG, NW, SEG // NW // CH, CH).astype(jnp.int32)
    tt0 = token_type_emb[0]

    rows = [_gather_seg(ids[k], word_emb) for k in range(NSEG)]
    emb_flat = None
    for k in range(NSEG):
        emb_flat = _ln_tc_seg(rows[k], tt0, ln_gamma, ln_beta, k, emb_flat)
    embeddings = emb_flat.reshape(b, s, HID)

    cos_c, sin_c = pl.pallas_call(
        _rope_body,
        out_shape=(
            jax.ShapeDtypeStruct((s, HEAD_DIM), jnp.float32),
            jax.ShapeDtypeStruct((s, HEAD_DIM), jnp.float32),
        ),
    )()
    rope_cos = jnp.broadcast_to(cos_c[None, :, None, :], (b, s, 1, HEAD_DIM))
    rope_sin = jnp.broadcast_to(sin_c[None, :, None, :], (b, s, 1, HEAD_DIM))

    attention_mask = jnp.ones((b, s), dtype=jnp.float32)
    return embeddings, attention_mask, rope_cos, rope_sin


# speedup vs baseline: 1.0262x; 1.0262x over previous
import functools
import math

import jax
import jax.numpy as jnp
from jax import lax
from jax.experimental import pallas as pl
from jax.experimental.pallas import tpu as pltpu
from jax.experimental.pallas import tpu_sc as plsc

HID = 1024
HEAD_DIM = 64
BASE = 10000.0
EPS = 1e-12

NC = 2
NS = 16
NW = NC * NS

TOK = 8192
TPW = TOK // NW
CH = 32
NCH = TPW // CH

_sc_mesh = plsc.VectorSubcoreMesh(
    core_axis_name="c", subcore_axis_name="s", num_cores=NC, num_subcores=NS
)


def _make_gather_sc(n_tok):
    tpw = n_tok // NW
    nch = tpw // CH

    @functools.partial(
        pl.kernel,
        out_type=jax.ShapeDtypeStruct((n_tok, HID), jnp.float32),
        mesh=_sc_mesh,
        scratch_types=[
            pltpu.VMEM((nch, CH), jnp.int32),
            pltpu.VMEM((CH, HID), jnp.float32),
            pltpu.VMEM((CH, HID), jnp.float32),
            pltpu.SemaphoreType.DMA,
            pltpu.SemaphoreType.DMA,
        ],
    )
    def _gather_sc(ids_hbm, table_hbm, out_hbm, idx_v, buf_a, buf_b, sem_a, sem_b):
        wid = lax.axis_index("s") * NC + lax.axis_index("c")
        pltpu.sync_copy(ids_hbm.at[wid], idx_v)
        bufs = (buf_a, buf_b)
        sems = (sem_a, sem_b)

        def start(c):
            pltpu.make_async_copy(
                table_hbm.at[idx_v.at[c]], bufs[c % 2], sems[c % 2]
            ).start()

        def wait(c):
            pltpu.make_async_copy(
                table_hbm.at[idx_v.at[c]], bufs[c % 2], sems[c % 2]
            ).wait()

        start(0)
        for c in range(nch):
            if c + 1 < nch:
                start(c + 1)
            wait(c)
            pltpu.sync_copy(bufs[c % 2], out_hbm.at[pl.ds(wid * tpw + c * CH, CH)])

    return _gather_sc


def _ln_math(x, g, b):
    mu = jnp.mean(x, axis=1, keepdims=True)
    xc = x - mu
    var = jnp.mean(xc * xc, axis=1, keepdims=True)
    return xc * lax.rsqrt(var + EPS) * g + b


def _ln_body(rows_ref, tt_ref, g_ref, b_ref, out_ref):
    out_ref[...] = _ln_math(rows_ref[...] + tt_ref[...], g_ref[...], b_ref[...])


def _ln_body_acc(rows_ref, tt_ref, g_ref, b_ref, prev_ref, out_ref):
    del prev_ref
    out_ref[...] = _ln_math(rows_ref[...] + tt_ref[...], g_ref[...], b_ref[...])


TB = 2048


def _ln_tc_seg(rows, tt0, gamma, beta, seg, prev):
    n_tok = rows.shape[0]
    nblk = n_tok // TB
    off = seg * nblk
    row_spec = pl.BlockSpec((TB, HID), lambda i: (i, 0))
    chan_spec = pl.BlockSpec((1, HID), lambda i: (0, 0))
    out_spec = pl.BlockSpec((TB, HID), lambda i, o=off: (o + i, 0))
    args = [rows, tt0.reshape(1, HID), gamma.reshape(1, HID), beta.reshape(1, HID)]
    in_specs = [row_spec, chan_spec, chan_spec, chan_spec]
    body = _ln_body
    kwargs = {}
    if seg > 0:
        args.append(prev)
        in_specs.append(pl.BlockSpec(memory_space=pl.ANY))
        body = _ln_body_acc
        kwargs["input_output_aliases"] = {4: 0}
    return pl.pallas_call(
        body,
        grid=(nblk,),
        in_specs=in_specs,
        out_specs=out_spec,
        out_shape=jax.ShapeDtypeStruct((TOK, HID), jnp.float32),
        **kwargs,
    )(*args)


def _rope_body(cos_ref, sin_ref):
    s_len, d = cos_ref.shape
    half = d // 2
    pos = lax.broadcasted_iota(jnp.int32, (s_len, half), 0).astype(jnp.float32)
    i = lax.broadcasted_iota(jnp.int32, (s_len, half), 1).astype(jnp.float32)
    inv_freq = jnp.exp(i * (-2.0 * math.log(BASE) / d))
    ang = pos * inv_freq
    c = jnp.cos(ang)
    s = jnp.sin(ang)
    cos_ref[:, :half] = c
    cos_ref[:, half:] = c
    sin_ref[:, :half] = s
    sin_ref[:, half:] = s


NSEG = 1
SEG = TOK // NSEG
_gather_seg = _make_gather_sc(SEG)


def kernel(input_ids, word_emb, token_type_emb, ln_gamma, ln_beta):
    b, s = input_ids.shape
    ids = input_ids.reshape(NSEG, NW, SEG // NW // CH, CH).astype(jnp.int32)
    tt0 = token_type_emb[0]

    rows = [_gather_seg(ids[k], word_emb) for k in range(NSEG)]
    emb_flat = None
    for k in range(NSEG):
        emb_flat = _ln_tc_seg(rows[k], tt0, ln_gamma, ln_beta, k, emb_flat)
    embeddings = emb_flat.reshape(b, s, HID)

    cos_c, sin_c = pl.pallas_call(
        _rope_body,
        out_shape=(
            jax.ShapeDtypeStruct((s, HEAD_DIM), jnp.float32),
            jax.ShapeDtypeStruct((s, HEAD_DIM), jnp.float32),
        ),
    )()
    rope_cos = jnp.broadcast_to(cos_c[None, :, None, :], (b, s, 1, HEAD_DIM))
    rope_sin = jnp.broadcast_to(sin_c[None, :, None, :], (b, s, 1, HEAD_DIM))

    attention_mask = jnp.ones((b, s), dtype=jnp.float32)
    return embeddings, attention_mask, rope_cos, rope_sin
